# Initial kernel scaffold; baseline (speedup 1.0000x reference)
import jax
import jax.numpy as jnp
from jax.experimental import pallas as pl


def kernel(x, edge_index, W, b):
    raise NotImplementedError("write your pallas kernel here")


# speedup vs baseline: 8.8756x; 8.8756x over previous
import functools

import jax
import jax.numpy as jnp
from jax import lax
from jax.experimental import pallas as pl
from jax.experimental.pallas import tpu as pltpu
from jax.experimental.pallas import tpu_sc as plsc

N = 10000
NP = 10240
D = 128
DH = 64
E = 320000
NC, NS = 2, 16
NW = NC * NS
EPT = 10240
CH = 128
NCH = EPT // CH
EP = NW * EPT
RPT = NP // NW
RPS = NP // NS
PAD = NP - 1

_mesh = plsc.VectorSubcoreMesh(
    core_axis_name="c", subcore_axis_name="s", num_cores=NC, num_subcores=NS
)
_params = pltpu.CompilerParams(
    needs_layout_passes=False, use_tc_tiling_on_sc=False)


def _worker_id():
    return lax.axis_index("s") * NC + lax.axis_index("c")


def _mm_body(x_ref, w_ref, o_ref):
    o_ref[0] = lax.dot_general(
        x_ref[...], w_ref[...], (((1,), (1,)), ((), ())),
        preferred_element_type=jnp.float32,
    )


def _matmul(x_pad, w):
    return pl.pallas_call(
        _mm_body,
        grid=(2, NP // 1024),
        in_specs=[
            pl.BlockSpec((1024, D), lambda h, i: (i, 0)),
            pl.BlockSpec((DH, D), lambda h, i: (h, 0)),
        ],
        out_specs=pl.BlockSpec((1, 1024, DH), lambda h, i: (h, i, 0)),
        out_shape=jax.ShapeDtypeStruct((2, NP, DH), jnp.float32),
    )(x_pad, w)


@functools.partial(
    pl.kernel,
    out_type=jax.ShapeDtypeStruct((NC * NP,), jnp.float32),
    mesh=_mesh,
    scratch_types=[
        pltpu.VMEM((EPT,), jnp.int32),
        pltpu.VMEM((NP,), jnp.float32),
        pltpu.VMEM((NS, RPS), jnp.float32),
        pltpu.VMEM((RPS,), jnp.float32),
        pltpu.VMEM_SHARED((NS, NP), jnp.float32),
    ],
    compiler_params=_params,
)
def _deg_kernel(dst_hbm, out_hbm, idx_v, hist_v, red_v, sum_v, sh):
    c = lax.axis_index("c")
    s = lax.axis_index("s")
    wid = s * NC + c
    pltpu.sync_copy(dst_hbm.at[wid], idx_v)
    zeros = jnp.zeros((16,), jnp.float32)

    @pl.loop(0, NP // 16)
    def _(i):
        hist_v[pl.ds(i * 16, 16)] = zeros

    ones = jnp.ones((16,), jnp.float32)

    @pl.loop(0, EPT // 16)
    def _(i):
        ii = idx_v[pl.ds(i * 16, 16)]
        plsc.addupdate_scatter(hist_v, [ii], ones)

    pltpu.sync_copy(hist_v, sh.at[s])
    plsc.subcore_barrier()
    pltpu.sync_copy(sh.at[:, pl.ds(s * RPS, RPS)], red_v)

    @pl.loop(0, RPS // 16)
    def _(k):
        sl = pl.ds(k * 16, 16)
        acc = red_v[0, sl]
        for t in range(1, NS):
            acc = acc + red_v[t, sl]
        sum_v[sl] = acc

    pltpu.sync_copy(sum_v, out_hbm.at[pl.ds(c * NP + s * RPS, RPS)])


@functools.partial(
    pl.kernel,
    out_type=(
        jax.ShapeDtypeStruct((2, NP, DH), jnp.float32),
        jax.ShapeDtypeStruct((NP,), jnp.float32),
        jax.ShapeDtypeStruct((NP,), jnp.float32),
    ),
    mesh=_mesh,
    scratch_types=[
        pltpu.VMEM((RPT,), jnp.float32),
        pltpu.VMEM((RPT,), jnp.float32),
        pltpu.VMEM((RPT,), jnp.float32),
        pltpu.VMEM((RPT,), jnp.float32),
        pltpu.VMEM((RPT, DH), jnp.float32),
        pltpu.VMEM((RPT, DH), jnp.float32),
    ],
    compiler_params=_params,
)
def _scale0_kernel(hist_hbm, z_hbm, zs_hbm, dis_hbm, dinv_hbm,
                   hv0, hv1, dis_v, dinv_v, zb0, zb1):
    wid = _worker_id()
    r0 = wid * RPT
    pltpu.sync_copy(hist_hbm.at[pl.ds(r0, RPT)], hv0)
    pltpu.sync_copy(hist_hbm.at[pl.ds(NP + r0, RPT)], hv1)
    pltpu.sync_copy(z_hbm.at[0, pl.ds(r0, RPT)], zb0)
    pltpu.sync_copy(z_hbm.at[1, pl.ds(r0, RPT)], zb1)

    @pl.loop(0, RPT // 16)
    def _(k):
        s = pl.ds(k * 16, 16)
        deg = jnp.ones((16,), jnp.float32) + hv0[s] + hv1[s]
        dinv_v[s] = 1.0 / deg
        xi = lax.bitcast_convert_type(deg, jnp.int32)
        yi = jnp.int32(0x5F3759DF) - (xi >> 1)
        y = lax.bitcast_convert_type(yi, jnp.float32)
        for _ in range(3):
            y = y * (1.5 - 0.5 * deg * y * y)
        dis_v[s] = y

    pltpu.sync_copy(dis_v, dis_hbm.at[pl.ds(r0, RPT)])
    pltpu.sync_copy(dinv_v, dinv_hbm.at[pl.ds(r0, RPT)])

    @pl.loop(0, RPT)
    def _(n):
        dn = plsc.load_gather(dis_v, [jnp.full((16,), 0, jnp.int32) + n])
        for k in range(DH // 16):
            s = pl.ds(k * 16, 16)
            zb0[n, s] = zb0[n, s] * dn
            zb1[n, s] = zb1[n, s] * dn

    pltpu.sync_copy(zb0, zs_hbm.at[0, pl.ds(r0, RPT)])
    pltpu.sync_copy(zb1, zs_hbm.at[1, pl.ds(r0, RPT)])


@functools.partial(
    pl.kernel,
    out_type=jax.ShapeDtypeStruct((NC, 2, NP, DH), jnp.float32),
    mesh=_mesh,
    scratch_types=[
        pltpu.VMEM((NCH, CH), jnp.int32),
        pltpu.VMEM((NCH, CH), jnp.int32),
        pltpu.VMEM((2, CH, DH), jnp.float32),
        pltpu.VMEM_SHARED((NP, DH), jnp.float32),
        pltpu.SemaphoreType.DMA,
    ],
    compiler_params=_params,
)
def _prop_kernel(table_hbm, src_hbm, dst_hbm, out_hbm,
                 src_v, dst_v, rows_v, acc_sh, gsem):
    c = lax.axis_index("c")
    s = lax.axis_index("s")
    wid = s * NC + c
    pltpu.sync_copy(src_hbm.at[wid], src_v)
    pltpu.sync_copy(dst_hbm.at[wid], dst_v)

    zeros = jnp.zeros((16,), jnp.float32)

    for h in range(2):
        @pl.loop(0, CH * DH // 16)
        def _(i):
            r = i // (DH // 16)
            k = i % (DH // 16)
            rows_v[0, r, pl.ds(k * 16, 16)] = zeros

        @pl.loop(0, RPS // CH)
        def _(j):
            pltpu.sync_copy(rows_v.at[0],
                            acc_sh.at[pl.ds(s * RPS + j * CH, CH)])

        plsc.subcore_barrier()

        @pl.loop(0, NCH // 8)
        def _(jj):
            base = jj * 8
            cp = pltpu.async_copy(
                table_hbm.at[h].at[src_v.at[base]], rows_v.at[0], gsem)
            for b in range(8):
                cp.wait()
                if b < 7:
                    cp = pltpu.async_copy(
                        table_hbm.at[h].at[src_v.at[base + b + 1]],
                        rows_v.at[(b + 1) % 2], gsem)
                pltpu.sync_copy(rows_v.at[b % 2],
                                acc_sh.at[dst_v.at[base + b]], add=True)

        plsc.subcore_barrier()
        pltpu.sync_copy(acc_sh.at[pl.ds(s * RPS, RPS)],
                        out_hbm.at[c, h, pl.ds(s * RPS, RPS)])


@functools.partial(
    pl.kernel,
    out_type=jax.ShapeDtypeStruct((2, NP, DH), jnp.float32),
    mesh=_mesh,
    scratch_types=[
        pltpu.VMEM((64, DH), jnp.float32),
        pltpu.VMEM((64, DH), jnp.float32),
        pltpu.VMEM((64, DH), jnp.float32),
        pltpu.VMEM((RPT,), jnp.float32),
    ],
    compiler_params=_params,
)
def _combine_mid(q_hbm, base_hbm, scale_hbm, out_hbm, av, bv, cv, sc_v):
    wid = _worker_id()
    r0 = wid * RPT
    pltpu.sync_copy(scale_hbm.at[pl.ds(r0, RPT)], sc_v)

    for h in range(2):
        @pl.loop(0, RPT // 64)
        def _(cc):
            rb = r0 + cc * 64
            pltpu.sync_copy(q_hbm.at[0, h, pl.ds(rb, 64)], av)
            pltpu.sync_copy(q_hbm.at[1, h, pl.ds(rb, 64)], bv)
            pltpu.sync_copy(base_hbm.at[h, pl.ds(rb, 64)], cv)

            @pl.loop(0, 64)
            def _(n):
                dn = plsc.load_gather(
                    sc_v, [jnp.full((16,), 0, jnp.int32) + (cc * 64 + n)])
                for k in range(DH // 16):
                    sl = pl.ds(k * 16, 16)
                    av[n, sl] = (av[n, sl] + bv[n, sl] + cv[n, sl]) * dn

            pltpu.sync_copy(av, out_hbm.at[h, pl.ds(rb, 64)])


@functools.partial(
    pl.kernel,
    out_type=jax.ShapeDtypeStruct((NP, D), jnp.float32),
    mesh=_mesh,
    scratch_types=[
        pltpu.VMEM((64, DH), jnp.float32),
        pltpu.VMEM((64, DH), jnp.float32),
        pltpu.VMEM((64, DH), jnp.float32),
        pltpu.VMEM((64, D), jnp.float32),
        pltpu.VMEM((RPT,), jnp.float32),
        pltpu.VMEM((D,), jnp.float32),
    ],
    compiler_params=_params,
)
def _combine_final(q_hbm, base_hbm, scale_hbm, b_hbm, out_hbm,
                   av, bv, cv, wv, sc_v, bb):
    wid = _worker_id()
    r0 = wid * RPT
    pltpu.sync_copy(scale_hbm.at[pl.ds(r0, RPT)], sc_v)
    pltpu.sync_copy(b_hbm, bb)

    @pl.loop(0, RPT // 64)
    def _(cc):
        rb = r0 + cc * 64
        for h in range(2):
            pltpu.sync_copy(q_hbm.at[0, h, pl.ds(rb, 64)], av)
            pltpu.sync_copy(q_hbm.at[1, h, pl.ds(rb, 64)], bv)
            pltpu.sync_copy(base_hbm.at[h, pl.ds(rb, 64)], cv)

            @pl.loop(0, 64)
            def _(n):
                dn = plsc.load_gather(
                    sc_v, [jnp.full((16,), 0, jnp.int32) + (cc * 64 + n)])
                for k in range(DH // 16):
                    sl = pl.ds(k * 16, 16)
                    wv[n, pl.ds(h * DH + k * 16, 16)] = (
                        (av[n, sl] + bv[n, sl] + cv[n, sl]) * dn
                        + bb[pl.ds(h * DH + k * 16, 16)])

        pltpu.sync_copy(wv, out_hbm.at[pl.ds(rb, 64)])


def kernel(x, edge_index, W, b):
    x = x.astype(jnp.float32)
    src = edge_index[0].astype(jnp.int32)
    dst = edge_index[1].astype(jnp.int32)
    pad_e = EP - E
    src_p = jnp.concatenate(
        [src, jnp.full((pad_e,), PAD, jnp.int32)]).reshape(NW, NCH, CH)
    dst_p = jnp.concatenate(
        [dst, jnp.full((pad_e,), PAD, jnp.int32)]).reshape(NW, NCH, CH)
    dst_flat = dst_p.reshape(NW, EPT)
    x_pad = jnp.concatenate([x, jnp.zeros((NP - N, D), jnp.float32)], axis=0)

    z = _matmul(x_pad, W)
    hist = _deg_kernel(dst_flat)
    zs, dis, dinv = _scale0_kernel(hist, z)
    q = _prop_kernel(zs, src_p, dst_p)
    g = _combine_mid(q, zs, dinv)
    r = _prop_kernel(g, src_p, dst_p)
    out = _combine_final(r, g, dis, b)
    return out[:N]


# speedup vs baseline: 9.6936x; 1.0922x over previous
import functools

import jax
import jax.numpy as jnp
from jax import lax
from jax.experimental import pallas as pl
from jax.experimental.pallas import tpu as pltpu
from jax.experimental.pallas import tpu_sc as plsc

N = 10000
NP = 10240
D = 128
DH = 64
E = 320000
NC, NS = 2, 16
NW = NC * NS
EPT = 10240
CH = 128
NCH = EPT // CH
EP = NW * EPT
RPT = NP // NW
RPS = NP // NS
PAD = NP - 1

_mesh = plsc.VectorSubcoreMesh(
    core_axis_name="c", subcore_axis_name="s", num_cores=NC, num_subcores=NS
)
_params = pltpu.CompilerParams(
    needs_layout_passes=False, use_tc_tiling_on_sc=False)


def _worker_id():
    return lax.axis_index("s") * NC + lax.axis_index("c")


def _mm_body(x_ref, w_ref, o_ref):
    o_ref[0] = lax.dot_general(
        x_ref[...], w_ref[...], (((1,), (1,)), ((), ())),
        preferred_element_type=jnp.float32,
    )


def _matmul(x_pad, w):
    return pl.pallas_call(
        _mm_body,
        grid=(2, NP // 1024),
        in_specs=[
            pl.BlockSpec((1024, D), lambda h, i: (i, 0)),
            pl.BlockSpec((DH, D), lambda h, i: (h, 0)),
        ],
        out_specs=pl.BlockSpec((1, 1024, DH), lambda h, i: (h, i, 0)),
        out_shape=jax.ShapeDtypeStruct((2, NP, DH), jnp.float32),
    )(x_pad, w)


@functools.partial(
    pl.kernel,
    out_type=jax.ShapeDtypeStruct((NC * NP,), jnp.float32),
    mesh=_mesh,
    scratch_types=[
        pltpu.VMEM((EPT,), jnp.int32),
        pltpu.VMEM((NP,), jnp.float32),
        pltpu.VMEM((NS, RPS), jnp.float32),
        pltpu.VMEM((RPS,), jnp.float32),
        pltpu.VMEM_SHARED((NS, NP), jnp.float32),
    ],
    compiler_params=_params,
)
def _deg_kernel(dst_hbm, out_hbm, idx_v, hist_v, red_v, sum_v, sh):
    c = lax.axis_index("c")
    s = lax.axis_index("s")
    wid = s * NC + c
    pltpu.sync_copy(dst_hbm.at[wid], idx_v)
    zeros = jnp.zeros((16,), jnp.float32)

    @pl.loop(0, NP // 16)
    def _(i):
        hist_v[pl.ds(i * 16, 16)] = zeros

    ones = jnp.ones((16,), jnp.float32)

    @pl.loop(0, EPT // 16)
    def _(i):
        ii = idx_v[pl.ds(i * 16, 16)]
        plsc.addupdate_scatter(hist_v, [ii], ones)

    pltpu.sync_copy(hist_v, sh.at[s])
    plsc.subcore_barrier()
    pltpu.sync_copy(sh.at[:, pl.ds(s * RPS, RPS)], red_v)

    @pl.loop(0, RPS // 16)
    def _(k):
        sl = pl.ds(k * 16, 16)
        acc = red_v[0, sl]
        for t in range(1, NS):
            acc = acc + red_v[t, sl]
        sum_v[sl] = acc

    pltpu.sync_copy(sum_v, out_hbm.at[pl.ds(c * NP + s * RPS, RPS)])


@functools.partial(
    pl.kernel,
    out_type=(
        jax.ShapeDtypeStruct((2, NP, DH), jnp.float32),
        jax.ShapeDtypeStruct((NP,), jnp.float32),
        jax.ShapeDtypeStruct((NP,), jnp.float32),
    ),
    mesh=_mesh,
    scratch_types=[
        pltpu.VMEM((RPT,), jnp.float32),
        pltpu.VMEM((RPT,), jnp.float32),
        pltpu.VMEM((RPT,), jnp.float32),
        pltpu.VMEM((RPT,), jnp.float32),
        pltpu.VMEM((RPT, DH), jnp.float32),
        pltpu.VMEM((RPT, DH), jnp.float32),
    ],
    compiler_params=_params,
)
def _scale0_kernel(hist_hbm, z_hbm, zs_hbm, dis_hbm, dinv_hbm,
                   hv0, hv1, dis_v, dinv_v, zb0, zb1):
    wid = _worker_id()
    r0 = wid * RPT
    pltpu.sync_copy(hist_hbm.at[pl.ds(r0, RPT)], hv0)
    pltpu.sync_copy(hist_hbm.at[pl.ds(NP + r0, RPT)], hv1)
    pltpu.sync_copy(z_hbm.at[0, pl.ds(r0, RPT)], zb0)
    pltpu.sync_copy(z_hbm.at[1, pl.ds(r0, RPT)], zb1)

    @pl.loop(0, RPT // 16)
    def _(k):
        s = pl.ds(k * 16, 16)
        deg = jnp.ones((16,), jnp.float32) + hv0[s] + hv1[s]
        dinv_v[s] = 1.0 / deg
        xi = lax.bitcast_convert_type(deg, jnp.int32)
        yi = jnp.int32(0x5F3759DF) - (xi >> 1)
        y = lax.bitcast_convert_type(yi, jnp.float32)
        for _ in range(3):
            y = y * (1.5 - 0.5 * deg * y * y)
        dis_v[s] = y

    pltpu.sync_copy(dis_v, dis_hbm.at[pl.ds(r0, RPT)])
    pltpu.sync_copy(dinv_v, dinv_hbm.at[pl.ds(r0, RPT)])

    @pl.loop(0, RPT)
    def _(n):
        dn = plsc.load_gather(dis_v, [jnp.full((16,), 0, jnp.int32) + n])
        for k in range(DH // 16):
            s = pl.ds(k * 16, 16)
            zb0[n, s] = zb0[n, s] * dn
            zb1[n, s] = zb1[n, s] * dn

    pltpu.sync_copy(zb0, zs_hbm.at[0, pl.ds(r0, RPT)])
    pltpu.sync_copy(zb1, zs_hbm.at[1, pl.ds(r0, RPT)])


@functools.partial(
    pl.kernel,
    out_type=jax.ShapeDtypeStruct((NC, 2, NP, DH), jnp.float32),
    mesh=_mesh,
    scratch_types=[
        pltpu.VMEM((NCH, CH), jnp.int32),
        pltpu.VMEM((NCH, CH), jnp.int32),
        pltpu.VMEM((4, CH, DH), jnp.float32),
        pltpu.VMEM((CH, DH), jnp.float32),
        pltpu.VMEM_SHARED((NP, DH), jnp.float32),
        pltpu.SemaphoreType.DMA,
        pltpu.SemaphoreType.DMA,
    ],
    compiler_params=_params,
)
def _prop_kernel(table_hbm, src_hbm, dst_hbm, out_hbm,
                 src_v, dst_v, rows_v, zb_v, acc_sh, gsem, ssem):
    c = lax.axis_index("c")
    s = lax.axis_index("s")
    wid = s * NC + c
    pltpu.sync_copy(src_hbm.at[wid], src_v)
    pltpu.sync_copy(dst_hbm.at[wid], dst_v)

    zeros = jnp.zeros((16,), jnp.float32)

    @pl.loop(0, CH * DH // 16)
    def _(i):
        r = i // (DH // 16)
        k = i % (DH // 16)
        zb_v[r, pl.ds(k * 16, 16)] = zeros

    for h in range(2):
        @pl.loop(0, RPS // CH)
        def _(j):
            pltpu.sync_copy(zb_v, acc_sh.at[pl.ds(s * RPS + j * CH, CH)])

        plsc.subcore_barrier()

        @pl.loop(0, NCH // 8)
        def _(jj):
            base = jj * 8
            g = [None] * 8
            sd = [None] * 8

            def _scatter(b):
                d = pltpu.make_async_copy(
                    rows_v.at[b % 4], acc_sh.at[dst_v.at[base + b]], ssem)
                d.start(add=True)
                return d

            for b in range(8):
                if b >= 4:
                    sd[b - 4].wait()
                g[b] = pltpu.make_async_copy(
                    table_hbm.at[h].at[src_v.at[base + b]],
                    rows_v.at[b % 4], gsem)
                g[b].start()
                if b >= 2:
                    g[b - 2].wait()
                    sd[b - 2] = _scatter(b - 2)
            for b in range(6, 8):
                g[b].wait()
                sd[b] = _scatter(b)
            for b in range(4, 8):
                sd[b].wait()

        plsc.subcore_barrier()
        pltpu.sync_copy(acc_sh.at[pl.ds(s * RPS, RPS)],
                        out_hbm.at[c, h, pl.ds(s * RPS, RPS)])


@functools.partial(
    pl.kernel,
    out_type=jax.ShapeDtypeStruct((2, NP, DH), jnp.float32),
    mesh=_mesh,
    scratch_types=[
        pltpu.VMEM((64, DH), jnp.float32),
        pltpu.VMEM((64, DH), jnp.float32),
        pltpu.VMEM((64, DH), jnp.float32),
        pltpu.VMEM((RPT,), jnp.float32),
    ],
    compiler_params=_params,
)
def _combine_mid(q_hbm, base_hbm, scale_hbm, out_hbm, av, bv, cv, sc_v):
    wid = _worker_id()
    r0 = wid * RPT
    pltpu.sync_copy(scale_hbm.at[pl.ds(r0, RPT)], sc_v)

    for h in range(2):
        @pl.loop(0, RPT // 64)
        def _(cc):
            rb = r0 + cc * 64
            pltpu.sync_copy(q_hbm.at[0, h, pl.ds(rb, 64)], av)
            pltpu.sync_copy(q_hbm.at[1, h, pl.ds(rb, 64)], bv)
            pltpu.sync_copy(base_hbm.at[h, pl.ds(rb, 64)], cv)

            @pl.loop(0, 64)
            def _(n):
                dn = plsc.load_gather(
                    sc_v, [jnp.full((16,), 0, jnp.int32) + (cc * 64 + n)])
                for k in range(DH // 16):
                    sl = pl.ds(k * 16, 16)
                    av[n, sl] = (av[n, sl] + bv[n, sl] + cv[n, sl]) * dn

            pltpu.sync_copy(av, out_hbm.at[h, pl.ds(rb, 64)])


@functools.partial(
    pl.kernel,
    out_type=jax.ShapeDtypeStruct((NP, D), jnp.float32),
    mesh=_mesh,
    scratch_types=[
        pltpu.VMEM((64, DH), jnp.float32),
        pltpu.VMEM((64, DH), jnp.float32),
        pltpu.VMEM((64, DH), jnp.float32),
        pltpu.VMEM((64, D), jnp.float32),
        pltpu.VMEM((RPT,), jnp.float32),
        pltpu.VMEM((D,), jnp.float32),
    ],
    compiler_params=_params,
)
def _combine_final(q_hbm, base_hbm, scale_hbm, b_hbm, out_hbm,
                   av, bv, cv, wv, sc_v, bb):
    wid = _worker_id()
    r0 = wid * RPT
    pltpu.sync_copy(scale_hbm.at[pl.ds(r0, RPT)], sc_v)
    pltpu.sync_copy(b_hbm, bb)

    @pl.loop(0, RPT // 64)
    def _(cc):
        rb = r0 + cc * 64
        for h in range(2):
            pltpu.sync_copy(q_hbm.at[0, h, pl.ds(rb, 64)], av)
            pltpu.sync_copy(q_hbm.at[1, h, pl.ds(rb, 64)], bv)
            pltpu.sync_copy(base_hbm.at[h, pl.ds(rb, 64)], cv)

            @pl.loop(0, 64)
            def _(n):
                dn = plsc.load_gather(
                    sc_v, [jnp.full((16,), 0, jnp.int32) + (cc * 64 + n)])
                for k in range(DH // 16):
                    sl = pl.ds(k * 16, 16)
                    wv[n, pl.ds(h * DH + k * 16, 16)] = (
                        (av[n, sl] + bv[n, sl] + cv[n, sl]) * dn
                        + bb[pl.ds(h * DH + k * 16, 16)])

        pltpu.sync_copy(wv, out_hbm.at[pl.ds(rb, 64)])


def kernel(x, edge_index, W, b):
    x = x.astype(jnp.float32)
    src = edge_index[0].astype(jnp.int32)
    dst = edge_index[1].astype(jnp.int32)
    pad_e = EP - E
    src_p = jnp.concatenate(
        [src, jnp.full((pad_e,), PAD, jnp.int32)]).reshape(NW, NCH, CH)
    dst_p = jnp.concatenate(
        [dst, jnp.full((pad_e,), PAD, jnp.int32)]).reshape(NW, NCH, CH)
    dst_flat = dst_p.reshape(NW, EPT)
    x_pad = jnp.concatenate([x, jnp.zeros((NP - N, D), jnp.float32)], axis=0)

    z = _matmul(x_pad, W)
    hist = _deg_kernel(dst_flat)
    zs, dis, dinv = _scale0_kernel(hist, z)
    q = _prop_kernel(zs, src_p, dst_p)
    g = _combine_mid(q, zs, dinv)
    r = _prop_kernel(g, src_p, dst_p)
    out = _combine_final(r, g, dis, b)
    return out[:N]


# speedup vs baseline: 9.8720x; 1.0184x over previous
import functools

import jax
import jax.numpy as jnp
from jax import lax
from jax.experimental import pallas as pl
from jax.experimental.pallas import tpu as pltpu
from jax.experimental.pallas import tpu_sc as plsc

N = 10000
NP = 10240
D = 128
DH = 64
E = 320000
NC, NS = 2, 16
NW = NC * NS
EPT = 10240
CH = 128
NCH = EPT // CH
EP = NW * EPT
RPT = NP // NW
RPS = NP // NS
PAD = NP - 1

_mesh = plsc.VectorSubcoreMesh(
    core_axis_name="c", subcore_axis_name="s", num_cores=NC, num_subcores=NS
)
_params = pltpu.CompilerParams(
    needs_layout_passes=False, use_tc_tiling_on_sc=False)


def _worker_id():
    return lax.axis_index("s") * NC + lax.axis_index("c")


def _mm_body(x_ref, w_ref, o_ref):
    o_ref[0] = lax.dot_general(
        x_ref[...], w_ref[...], (((1,), (1,)), ((), ())),
        preferred_element_type=jnp.float32,
    )


def _matmul(x_pad, w):
    return pl.pallas_call(
        _mm_body,
        grid=(2, NP // 1024),
        in_specs=[
            pl.BlockSpec((1024, D), lambda h, i: (i, 0)),
            pl.BlockSpec((DH, D), lambda h, i: (h, 0)),
        ],
        out_specs=pl.BlockSpec((1, 1024, DH), lambda h, i: (h, i, 0)),
        out_shape=jax.ShapeDtypeStruct((2, NP, DH), jnp.float32),
    )(x_pad, w)


@functools.partial(
    pl.kernel,
    out_type=jax.ShapeDtypeStruct((NC * NP,), jnp.float32),
    mesh=_mesh,
    scratch_types=[
        pltpu.VMEM((EPT,), jnp.int32),
        pltpu.VMEM((NP,), jnp.float32),
        pltpu.VMEM((NS, RPS), jnp.float32),
        pltpu.VMEM((RPS,), jnp.float32),
        pltpu.VMEM_SHARED((NS, NP), jnp.float32),
    ],
    compiler_params=_params,
)
def _deg_kernel(dst_hbm, out_hbm, idx_v, hist_v, red_v, sum_v, sh):
    c = lax.axis_index("c")
    s = lax.axis_index("s")
    wid = s * NC + c
    pltpu.sync_copy(dst_hbm.at[wid], idx_v)
    zeros = jnp.zeros((16,), jnp.float32)

    @pl.loop(0, NP // 16)
    def _(i):
        hist_v[pl.ds(i * 16, 16)] = zeros

    ones = jnp.ones((16,), jnp.float32)

    @pl.loop(0, EPT // 16)
    def _(i):
        ii = idx_v[pl.ds(i * 16, 16)]
        plsc.addupdate_scatter(hist_v, [ii], ones)

    pltpu.sync_copy(hist_v, sh.at[s])
    plsc.subcore_barrier()
    pltpu.sync_copy(sh.at[:, pl.ds(s * RPS, RPS)], red_v)

    @pl.loop(0, RPS // 16)
    def _(k):
        sl = pl.ds(k * 16, 16)
        acc = red_v[0, sl]
        for t in range(1, NS):
            acc = acc + red_v[t, sl]
        sum_v[sl] = acc

    pltpu.sync_copy(sum_v, out_hbm.at[pl.ds(c * NP + s * RPS, RPS)])


@functools.partial(
    pl.kernel,
    out_type=(
        jax.ShapeDtypeStruct((2, NP, DH), jnp.float32),
        jax.ShapeDtypeStruct((NP,), jnp.float32),
        jax.ShapeDtypeStruct((NP,), jnp.float32),
    ),
    mesh=_mesh,
    scratch_types=[
        pltpu.VMEM((RPT,), jnp.float32),
        pltpu.VMEM((RPT,), jnp.float32),
        pltpu.VMEM((RPT,), jnp.float32),
        pltpu.VMEM((RPT,), jnp.float32),
        pltpu.VMEM((RPT, DH), jnp.float32),
        pltpu.VMEM((RPT, DH), jnp.float32),
    ],
    compiler_params=_params,
)
def _scale0_kernel(hist_hbm, z_hbm, zs_hbm, dis_hbm, dinv_hbm,
                   hv0, hv1, dis_v, dinv_v, zb0, zb1):
    wid = _worker_id()
    r0 = wid * RPT
    pltpu.sync_copy(hist_hbm.at[pl.ds(r0, RPT)], hv0)
    pltpu.sync_copy(hist_hbm.at[pl.ds(NP + r0, RPT)], hv1)
    pltpu.sync_copy(z_hbm.at[0, pl.ds(r0, RPT)], zb0)
    pltpu.sync_copy(z_hbm.at[1, pl.ds(r0, RPT)], zb1)

    @pl.loop(0, RPT // 16)
    def _(k):
        s = pl.ds(k * 16, 16)
        deg = jnp.ones((16,), jnp.float32) + hv0[s] + hv1[s]
        dinv_v[s] = 1.0 / deg
        xi = lax.bitcast_convert_type(deg, jnp.int32)
        yi = jnp.int32(0x5F3759DF) - (xi >> 1)
        y = lax.bitcast_convert_type(yi, jnp.float32)
        for _ in range(3):
            y = y * (1.5 - 0.5 * deg * y * y)
        dis_v[s] = y

    pltpu.sync_copy(dis_v, dis_hbm.at[pl.ds(r0, RPT)])
    pltpu.sync_copy(dinv_v, dinv_hbm.at[pl.ds(r0, RPT)])

    @pl.loop(0, RPT)
    def _(n):
        dn = plsc.load_gather(dis_v, [jnp.full((16,), 0, jnp.int32) + n])
        for k in range(DH // 16):
            s = pl.ds(k * 16, 16)
            zb0[n, s] = zb0[n, s] * dn
            zb1[n, s] = zb1[n, s] * dn

    pltpu.sync_copy(zb0, zs_hbm.at[0, pl.ds(r0, RPT)])
    pltpu.sync_copy(zb1, zs_hbm.at[1, pl.ds(r0, RPT)])


@functools.partial(
    pl.kernel,
    out_type=jax.ShapeDtypeStruct((NC, 2, NP, DH), jnp.float32),
    mesh=_mesh,
    scratch_types=[
        pltpu.VMEM((NCH, CH), jnp.int32),
        pltpu.VMEM((NCH, CH), jnp.int32),
        pltpu.VMEM((4, CH, DH), jnp.float32),
        pltpu.VMEM((CH, DH), jnp.float32),
        pltpu.VMEM_SHARED((NP, DH), jnp.float32),
        pltpu.SemaphoreType.DMA,
        pltpu.SemaphoreType.DMA,
    ],
    compiler_params=_params,
)
def _prop_kernel(table_hbm, src_hbm, dst_hbm, out_hbm,
                 src_v, dst_v, rows_v, zb_v, acc_sh, gsem, ssem):
    c = lax.axis_index("c")
    s = lax.axis_index("s")
    wid = s * NC + c
    pltpu.sync_copy(src_hbm.at[wid], src_v)
    pltpu.sync_copy(dst_hbm.at[wid], dst_v)

    zeros = jnp.zeros((16,), jnp.float32)

    @pl.loop(0, CH * DH // 16)
    def _(i):
        r = i // (DH // 16)
        k = i % (DH // 16)
        zb_v[r, pl.ds(k * 16, 16)] = zeros

    for h in range(2):
        @pl.loop(0, RPS // CH)
        def _(j):
            pltpu.sync_copy(zb_v, acc_sh.at[pl.ds(s * RPS + j * CH, CH)])

        plsc.subcore_barrier()

        @pl.loop(0, NCH // 10)
        def _(jj):
            base = jj * 10
            g = [None] * 10
            sd = [None] * 10

            def _scatter(b):
                d = pltpu.make_async_copy(
                    rows_v.at[b % 4], acc_sh.at[dst_v.at[base + b]], ssem)
                d.start(add=True)
                return d

            for b in range(10):
                if b >= 4:
                    sd[b - 4].wait()
                g[b] = pltpu.make_async_copy(
                    table_hbm.at[h].at[src_v.at[base + b]],
                    rows_v.at[b % 4], gsem)
                g[b].start()
                if b >= 2:
                    g[b - 2].wait()
                    sd[b - 2] = _scatter(b - 2)
            for b in range(8, 10):
                g[b].wait()
                sd[b] = _scatter(b)
            for b in range(6, 10):
                sd[b].wait()

        plsc.subcore_barrier()
        pltpu.sync_copy(acc_sh.at[pl.ds(s * RPS, RPS)],
                        out_hbm.at[c, h, pl.ds(s * RPS, RPS)])


@functools.partial(
    pl.kernel,
    out_type=jax.ShapeDtypeStruct((2, NP, DH), jnp.float32),
    mesh=_mesh,
    scratch_types=[
        pltpu.VMEM((RPT, DH), jnp.float32),
        pltpu.VMEM((RPT, DH), jnp.float32),
        pltpu.VMEM((RPT, DH), jnp.float32),
        pltpu.VMEM((RPT,), jnp.float32),
    ],
    compiler_params=_params,
)
def _combine_mid(q_hbm, base_hbm, scale_hbm, out_hbm, av, bv, cv, sc_v):
    wid = _worker_id()
    r0 = wid * RPT
    pltpu.sync_copy(scale_hbm.at[pl.ds(r0, RPT)], sc_v)

    for h in range(2):
        pltpu.sync_copy(q_hbm.at[0, h, pl.ds(r0, RPT)], av)
        pltpu.sync_copy(q_hbm.at[1, h, pl.ds(r0, RPT)], bv)
        pltpu.sync_copy(base_hbm.at[h, pl.ds(r0, RPT)], cv)

        @pl.loop(0, RPT)
        def _(n):
            dn = plsc.load_gather(
                sc_v, [jnp.full((16,), 0, jnp.int32) + n])
            for k in range(DH // 16):
                sl = pl.ds(k * 16, 16)
                av[n, sl] = (av[n, sl] + bv[n, sl] + cv[n, sl]) * dn

        pltpu.sync_copy(av, out_hbm.at[h, pl.ds(r0, RPT)])


@functools.partial(
    pl.kernel,
    out_type=jax.ShapeDtypeStruct((NP, D), jnp.float32),
    mesh=_mesh,
    scratch_types=[
        pltpu.VMEM((RPT, DH), jnp.float32),
        pltpu.VMEM((RPT, DH), jnp.float32),
        pltpu.VMEM((RPT, DH), jnp.float32),
        pltpu.VMEM((RPT, D), jnp.float32),
        pltpu.VMEM((RPT,), jnp.float32),
        pltpu.VMEM((D,), jnp.float32),
    ],
    compiler_params=_params,
)
def _combine_final(q_hbm, base_hbm, scale_hbm, b_hbm, out_hbm,
                   av, bv, cv, wv, sc_v, bb):
    wid = _worker_id()
    r0 = wid * RPT
    pltpu.sync_copy(scale_hbm.at[pl.ds(r0, RPT)], sc_v)
    pltpu.sync_copy(b_hbm, bb)

    for h in range(2):
        pltpu.sync_copy(q_hbm.at[0, h, pl.ds(r0, RPT)], av)
        pltpu.sync_copy(q_hbm.at[1, h, pl.ds(r0, RPT)], bv)
        pltpu.sync_copy(base_hbm.at[h, pl.ds(r0, RPT)], cv)

        @pl.loop(0, RPT)
        def _(n):
            dn = plsc.load_gather(
                sc_v, [jnp.full((16,), 0, jnp.int32) + n])
            for k in range(DH // 16):
                sl = pl.ds(k * 16, 16)
                wv[n, pl.ds(h * DH + k * 16, 16)] = (
                    (av[n, sl] + bv[n, sl] + cv[n, sl]) * dn
                    + bb[pl.ds(h * DH + k * 16, 16)])

    pltpu.sync_copy(wv, out_hbm.at[pl.ds(r0, RPT)])


def kernel(x, edge_index, W, b):
    x = x.astype(jnp.float32)
    src = edge_index[0].astype(jnp.int32)
    dst = edge_index[1].astype(jnp.int32)
    pad_e = EP - E
    src_p = jnp.concatenate(
        [src, jnp.full((pad_e,), PAD, jnp.int32)]).reshape(NW, NCH, CH)
    dst_p = jnp.concatenate(
        [dst, jnp.full((pad_e,), PAD, jnp.int32)]).reshape(NW, NCH, CH)
    dst_flat = dst_p.reshape(NW, EPT)
    x_pad = jnp.concatenate([x, jnp.zeros((NP - N, D), jnp.float32)], axis=0)

    z = _matmul(x_pad, W)
    hist = _deg_kernel(dst_flat)
    zs, dis, dinv = _scale0_kernel(hist, z)
    q = _prop_kernel(zs, src_p, dst_p)
    g = _combine_mid(q, zs, dinv)
    r = _prop_kernel(g, src_p, dst_p)
    out = _combine_final(r, g, dis, b)
    return out[:N]
